# Initial kernel scaffold; baseline (speedup 1.0000x reference)
import jax
import jax.numpy as jnp
from jax.experimental import pallas as pl


def kernel(node, edge_index, eps_k):
    raise NotImplementedError("write your pallas kernel here")


# speedup vs baseline: 4.2229x; 4.2229x over previous
import functools

import jax
import jax.numpy as jnp
from jax import lax
from jax.experimental import pallas as pl
from jax.experimental.pallas import tpu as pltpu
from jax.experimental.pallas import tpu_sc as plsc

NC = 2
NS = 16
NW = NC * NS
K = 128


def _sc_scatter(node, src, dst, zeros, n_pad, chunks_per_tile):
    d = node.shape[1]
    rows_per_tile = n_pad // NS
    mesh = plsc.VectorSubcoreMesh(core_axis_name="c", subcore_axis_name="s")

    @functools.partial(
        pl.kernel,
        mesh=mesh,
        out_type=jax.ShapeDtypeStruct((NC, n_pad, d), jnp.float32),
        scratch_types=[
            pltpu.VMEM((K,), jnp.int32),
            pltpu.VMEM((K,), jnp.int32),
            pltpu.VMEM((K, d), jnp.float32),
            pltpu.VMEM_SHARED((n_pad, d), jnp.float32),
            pltpu.SemaphoreType.DMA,
        ],
    )
    def body(node_hbm, src_hbm, dst_hbm, zeros_hbm, out_hbm,
             idx_s, idx_d, rows, acc, sem):
        cid = lax.axis_index("c")
        sid = lax.axis_index("s")
        wid = sid * NC + cid

        r0 = sid * rows_per_tile
        pltpu.sync_copy(zeros_hbm.at[pl.ds(r0, rows_per_tile)],
                        acc.at[pl.ds(r0, rows_per_tile)])
        plsc.subcore_barrier()

        base_w = wid * (chunks_per_tile * K)

        def chunk(c, carry):
            base = base_w + c * K
            pltpu.sync_copy(src_hbm.at[pl.ds(base, K)], idx_s)
            pltpu.sync_copy(dst_hbm.at[pl.ds(base, K)], idx_d)
            pltpu.async_copy(node_hbm.at[idx_s], rows, sem).wait()
            pltpu.sync_copy(rows, acc.at[idx_d], add=True)
            return carry

        lax.fori_loop(0, chunks_per_tile, chunk, 0)
        plsc.subcore_barrier()

        pltpu.sync_copy(acc.at[pl.ds(r0, rows_per_tile)],
                        out_hbm.at[cid, pl.ds(r0, rows_per_tile)])

    return body(node, src, dst, zeros)


def _tc_combine(node, acc0, acc1, eps):
    n, d = node.shape
    blk = 2000
    grid = n // blk

    def body(eps_ref, node_ref, a0_ref, a1_ref, out_ref):
        scale = 1.0 + eps_ref[0]
        out_ref[...] = scale * node_ref[...] + a0_ref[...] + a1_ref[...]

    return pl.pallas_call(
        body,
        grid=(grid,),
        in_specs=[
            pl.BlockSpec(memory_space=pltpu.SMEM),
            pl.BlockSpec((blk, d), lambda i: (i, 0)),
            pl.BlockSpec((blk, d), lambda i: (i, 0)),
            pl.BlockSpec((blk, d), lambda i: (i, 0)),
        ],
        out_specs=pl.BlockSpec((blk, d), lambda i: (i, 0)),
        out_shape=jax.ShapeDtypeStruct((n, d), jnp.float32),
    )(eps, node, acc0, acc1)


def kernel(node, edge_index, eps_k):
    n, d = node.shape
    m = edge_index.shape[1]
    src = edge_index[1]
    dst = edge_index[0]

    chunks_per_tile = -(-m // (NW * K))
    m_pad = NW * K * chunks_per_tile
    n_pad = -(-(n + 1) // (NS * 8)) * (NS * 8)

    if m_pad > m:
        pad = m_pad - m
        src = jnp.concatenate([src, jnp.zeros((pad,), jnp.int32)])
        dst = jnp.concatenate([dst, jnp.full((pad,), n, jnp.int32)])

    zeros = jnp.zeros((n_pad, d), jnp.float32)
    acc = _sc_scatter(node, src, dst, zeros, n_pad, chunks_per_tile)
    return _tc_combine(node, acc[0, :n], acc[1, :n],
                       jnp.reshape(eps_k, (1,)))
